# baseline (speedup 1.0000x reference)
import jax
import jax.numpy as jnp
from jax import lax
from jax.experimental import pallas as pl

_P = 16
_HB = 2


def _tree_sum(a):
    a = a[0:8] + a[8:16]
    a = a[0:4] + a[4:8]
    a = a[0:2] + a[2:4]
    return a[0] + a[1]


def _stats_kernel(inp_ref, rec_ref, out_ref):
    S = inp_ref.shape[2]
    P = _P
    Fp = S // P

    r_iota = lax.broadcasted_iota(jnp.int32, (Fp, S), 1)
    g_iota = lax.broadcasted_iota(jnp.int32, (Fp, S), 0)
    MwT = (r_iota // P == g_iota).astype(jnp.float32)
    Md = jnp.transpose(MwT)

    def pool(v):
        t = jnp.dot(MwT, v, preferred_element_type=jnp.float32)
        return jnp.dot(t, Md, preferred_element_type=jnp.float32)

    n = float(P * P * P)
    for h in range(_HB):
        x = inp_ref[0, h * P : (h + 1) * P]
        y = rec_ref[0, h * P : (h + 1) * P]
        s1 = pool(_tree_sum(x))
        s2 = pool(_tree_sum(x * x))
        r1 = pool(_tree_sum(y))
        r2 = pool(_tree_sum(y * y))
        c = pool(_tree_sum(x * y))

        mean = s1 / n
        var = s2 / n - mean * mean
        std = jnp.sqrt(var + 1e-6)
        l2 = r2 / n - 2.0 * (c / n - mean * (r1 / n)) / std + var / (var + 1e-6)
        out_ref[0, h] = l2


def _loss_kernel(scores_ref, l2_ref, out_ref):
    B, L = scores_ref.shape
    len_keep = int(round(L * (1.0 - 0.6)))
    i_iota = lax.broadcasted_iota(jnp.int32, (L, L), 0)
    j_iota = lax.broadcasted_iota(jnp.int32, (L, L), 1)
    num = jnp.zeros((1, 1), jnp.float32)
    den = jnp.zeros((1, 1), jnp.float32)
    for b in range(B):
        row = scores_ref[b : b + 1, :]
        col = jnp.transpose(row)
        less = row < col
        tie = (row == col) & (j_iota < i_iota)
        rank = jnp.sum((less | tie).astype(jnp.float32), axis=1, keepdims=True)
        nonact = (rank >= float(len_keep)).astype(jnp.float32)
        l2row = l2_ref[b : b + 1, :]
        num = num + jnp.dot(l2row, nonact, preferred_element_type=jnp.float32)
        den = den + jnp.sum(nonact)
    out_ref[:, :] = num / (den + 1e-8)


def kernel(inp_bchwd, rec_bchwd, scores):
    B, C, S = inp_bchwd.shape[0], inp_bchwd.shape[1], inp_bchwd.shape[2]
    P = _P
    Fp = S // P
    L = Fp * Fp * Fp

    inp3 = inp_bchwd.reshape(B, S, S, S)
    rec3 = rec_bchwd.reshape(B, S, S, S)

    l2 = pl.pallas_call(
        _stats_kernel,
        grid=(B, Fp // _HB),
        in_specs=[
            pl.BlockSpec((1, _HB * P, S, S), lambda b, h: (b, h, 0, 0)),
            pl.BlockSpec((1, _HB * P, S, S), lambda b, h: (b, h, 0, 0)),
        ],
        out_specs=pl.BlockSpec((1, _HB, Fp, Fp), lambda b, h: (b, h, 0, 0)),
        out_shape=jax.ShapeDtypeStruct((B, Fp, Fp, Fp), jnp.float32),
    )(inp3, rec3)

    l2 = l2.reshape(B, L)

    loss = pl.pallas_call(
        _loss_kernel,
        out_shape=jax.ShapeDtypeStruct((1, 1), jnp.float32),
    )(scores, l2)

    return loss[0, 0]


# speedup vs baseline: 1.0818x; 1.0818x over previous
import jax
import jax.numpy as jnp
from jax import lax
from jax.experimental import pallas as pl

_P = 16
_HB = 8


def _tree_sum(a):
    a = a[0:8] + a[8:16]
    a = a[0:4] + a[4:8]
    a = a[0:2] + a[2:4]
    return a[0] + a[1]


def _stats_kernel(inp_ref, rec_ref, out_ref):
    S = inp_ref.shape[2]
    P = _P
    Fp = S // P

    r_iota = lax.broadcasted_iota(jnp.int32, (Fp, S), 1)
    g_iota = lax.broadcasted_iota(jnp.int32, (Fp, S), 0)
    MwT = (r_iota // P == g_iota).astype(jnp.float32)
    Md = jnp.transpose(MwT)

    def pool(v):
        t = jnp.dot(MwT, v, preferred_element_type=jnp.float32)
        return jnp.dot(t, Md, preferred_element_type=jnp.float32)

    n = float(P * P * P)
    for h in range(_HB):
        x = inp_ref[0, h * P : (h + 1) * P]
        y = rec_ref[0, h * P : (h + 1) * P]
        s1 = pool(_tree_sum(x))
        s2 = pool(_tree_sum(x * x))
        r1 = pool(_tree_sum(y))
        r2 = pool(_tree_sum(y * y))
        c = pool(_tree_sum(x * y))

        mean = s1 / n
        var = s2 / n - mean * mean
        std = jnp.sqrt(var + 1e-6)
        l2 = r2 / n - 2.0 * (c / n - mean * (r1 / n)) / std + var / (var + 1e-6)
        out_ref[0, h] = l2


def _loss_kernel(scores_ref, l2_ref, out_ref):
    B, L = scores_ref.shape
    len_keep = int(round(L * (1.0 - 0.6)))
    i_iota = lax.broadcasted_iota(jnp.int32, (L, L), 0)
    j_iota = lax.broadcasted_iota(jnp.int32, (L, L), 1)
    num = jnp.zeros((1, 1), jnp.float32)
    den = jnp.zeros((1, 1), jnp.float32)
    for b in range(B):
        row = scores_ref[b : b + 1, :]
        col = jnp.transpose(row)
        less = row < col
        tie = (row == col) & (j_iota < i_iota)
        rank = jnp.sum((less | tie).astype(jnp.float32), axis=1, keepdims=True)
        nonact = (rank >= float(len_keep)).astype(jnp.float32)
        l2row = l2_ref[b : b + 1, :]
        num = num + jnp.dot(l2row, nonact, preferred_element_type=jnp.float32)
        den = den + jnp.sum(nonact)
    out_ref[:, :] = num / (den + 1e-8)


def kernel(inp_bchwd, rec_bchwd, scores):
    B, C, S = inp_bchwd.shape[0], inp_bchwd.shape[1], inp_bchwd.shape[2]
    P = _P
    Fp = S // P
    L = Fp * Fp * Fp

    inp3 = inp_bchwd.reshape(B, S, S, S)
    rec3 = rec_bchwd.reshape(B, S, S, S)

    l2 = pl.pallas_call(
        _stats_kernel,
        grid=(B, Fp // _HB),
        in_specs=[
            pl.BlockSpec((1, _HB * P, S, S), lambda b, h: (b, h, 0, 0)),
            pl.BlockSpec((1, _HB * P, S, S), lambda b, h: (b, h, 0, 0)),
        ],
        out_specs=pl.BlockSpec((1, _HB, Fp, Fp), lambda b, h: (b, h, 0, 0)),
        out_shape=jax.ShapeDtypeStruct((B, Fp, Fp, Fp), jnp.float32),
    )(inp3, rec3)

    l2 = l2.reshape(B, L)

    loss = pl.pallas_call(
        _loss_kernel,
        out_shape=jax.ShapeDtypeStruct((1, 1), jnp.float32),
    )(scores, l2)

    return loss[0, 0]


# speedup vs baseline: 1.0988x; 1.0157x over previous
import jax
import jax.numpy as jnp
from jax import lax
from jax.experimental import pallas as pl

_P = 16
_HB = 4


def _stats_kernel(inp_ref, rec_ref, out_ref):
    S = inp_ref.shape[2]
    P = _P
    Fp = S // P

    r_iota = lax.broadcasted_iota(jnp.int32, (Fp, S), 1)
    g_iota = lax.broadcasted_iota(jnp.int32, (Fp, S), 0)
    MwT = (r_iota // P == g_iota).astype(jnp.float32)
    Md = jnp.transpose(MwT)

    def pool(v):
        t = jnp.dot(MwT, v, preferred_element_type=jnp.float32)
        return jnp.dot(t, Md, preferred_element_type=jnp.float32)

    n = float(P * P * P)
    for h in range(_HB):
        s1 = s2 = r1 = r2 = c = None
        for k in range(P):
            xk = inp_ref[0, h * P + k]
            yk = rec_ref[0, h * P + k]
            if k == 0:
                s1, s2, r1, r2, c = xk, xk * xk, yk, yk * yk, xk * yk
            else:
                s1 = s1 + xk
                s2 = s2 + xk * xk
                r1 = r1 + yk
                r2 = r2 + yk * yk
                c = c + xk * yk
        s1, s2, r1, r2, c = pool(s1), pool(s2), pool(r1), pool(r2), pool(c)

        mean = s1 / n
        var = s2 / n - mean * mean
        std = jnp.sqrt(var + 1e-6)
        l2 = r2 / n - 2.0 * (c / n - mean * (r1 / n)) / std + var / (var + 1e-6)
        out_ref[0, h] = l2


def _loss_kernel(scores_ref, l2_ref, out_ref):
    B, L = scores_ref.shape
    len_keep = int(round(L * (1.0 - 0.6)))
    i_iota = lax.broadcasted_iota(jnp.int32, (L, L), 0)
    j_iota = lax.broadcasted_iota(jnp.int32, (L, L), 1)
    num = jnp.zeros((1, 1), jnp.float32)
    den = jnp.zeros((1, 1), jnp.float32)
    for b in range(B):
        row = scores_ref[b : b + 1, :]
        col = jnp.transpose(row)
        less = row < col
        tie = (row == col) & (j_iota < i_iota)
        rank = jnp.sum((less | tie).astype(jnp.float32), axis=1, keepdims=True)
        nonact = (rank >= float(len_keep)).astype(jnp.float32)
        l2row = l2_ref[b : b + 1, :]
        num = num + jnp.dot(l2row, nonact, preferred_element_type=jnp.float32)
        den = den + jnp.sum(nonact)
    out_ref[:, :] = num / (den + 1e-8)


def kernel(inp_bchwd, rec_bchwd, scores):
    B, C, S = inp_bchwd.shape[0], inp_bchwd.shape[1], inp_bchwd.shape[2]
    P = _P
    Fp = S // P
    L = Fp * Fp * Fp

    inp3 = inp_bchwd.reshape(B, S, S, S)
    rec3 = rec_bchwd.reshape(B, S, S, S)

    l2 = pl.pallas_call(
        _stats_kernel,
        grid=(B, Fp // _HB),
        in_specs=[
            pl.BlockSpec((1, _HB * P, S, S), lambda b, h: (b, h, 0, 0)),
            pl.BlockSpec((1, _HB * P, S, S), lambda b, h: (b, h, 0, 0)),
        ],
        out_specs=pl.BlockSpec((1, _HB, Fp, Fp), lambda b, h: (b, h, 0, 0)),
        out_shape=jax.ShapeDtypeStruct((B, Fp, Fp, Fp), jnp.float32),
    )(inp3, rec3)

    l2 = l2.reshape(B, L)

    loss = pl.pallas_call(
        _loss_kernel,
        out_shape=jax.ShapeDtypeStruct((1, 1), jnp.float32),
    )(scores, l2)

    return loss[0, 0]
